# baseline (speedup 1.0000x reference)
import functools

import jax
import jax.numpy as jnp
from jax import lax
from jax.experimental import pallas as pl
from jax.experimental.pallas import tpu as pltpu
from jax.experimental.pallas import tpu_sc as plsc

_DIM = 64
_B = 16384 * 26
_NC, _NS = 2, 16
_NW = _NC * _NS
_BPW = _B // _NW
_SUB = 128
_GRP = 512
_NSUB = _GRP // _SUB
_NGRP = _BPW // _GRP

_mesh = plsc.VectorSubcoreMesh(core_axis_name="c", subcore_axis_name="s")


@functools.partial(
    pl.kernel,
    mesh=_mesh,
    compiler_params=pltpu.CompilerParams(use_tc_tiling_on_sc=False),
    out_type=jax.ShapeDtypeStruct((_B, _DIM), jnp.float32),
    scratch_types=[
        pltpu.VMEM((_BPW,), jnp.int32),
        pltpu.VMEM((_GRP, _DIM), jnp.float32),
        pltpu.VMEM((_GRP, _DIM), jnp.float32),
        pltpu.SemaphoreType.DMA,
        pltpu.SemaphoreType.DMA,
        pltpu.SemaphoreType.DMA,
    ],
)
def _gather_kernel(table_hbm, idx_hbm, out_hbm, idx_v, buf0, buf1,
                   gsem, ssem0, ssem1):
    wid = lax.axis_index("s") * _NC + lax.axis_index("c")
    base = wid * _BPW
    pltpu.sync_copy(idx_hbm.at[pl.ds(base, _BPW)], idx_v)

    bufs = (buf0, buf1)
    ssems = (ssem0, ssem1)

    def gather_group(gg, buf):
        handles = []
        for j in range(_NSUB):
            idx_sl = idx_v.at[pl.ds(gg * _GRP + j * _SUB, _SUB)]
            handles.append(pltpu.async_copy(
                table_hbm.at[idx_sl], buf.at[pl.ds(j * _SUB, _SUB)], gsem))
        for h in handles:
            h.wait()

    def store_group(gg, buf, ssem):
        pltpu.async_copy(buf, out_hbm.at[pl.ds(base + gg * _GRP, _GRP)], ssem)

    def drain_store(buf, ssem):
        pltpu.make_async_copy(buf, out_hbm.at[pl.ds(base, _GRP)], ssem).wait()

    for p in (0, 1):
        gather_group(p, bufs[p])
        store_group(p, bufs[p], ssems[p])

    def pair_body(g, carry):
        for p in (0, 1):
            gg = 2 * g + p
            drain_store(bufs[p], ssems[p])
            gather_group(gg, bufs[p])
            store_group(gg, bufs[p], ssems[p])
        return carry

    lax.fori_loop(1, _NGRP // 2, pair_body, 0)

    for p in (0, 1):
        drain_store(bufs[p], ssems[p])


def kernel(idxs, features):
    flat = idxs.reshape(-1).astype(jnp.int32)
    out = _gather_kernel(features, flat)
    return out.reshape(idxs.shape + (features.shape[1],))


# speedup vs baseline: 1.0069x; 1.0069x over previous
import functools

import jax
import jax.numpy as jnp
from jax import lax
from jax.experimental import pallas as pl
from jax.experimental.pallas import tpu as pltpu
from jax.experimental.pallas import tpu_sc as plsc

_DIM = 64
_B = 16384 * 26
_NC, _NS = 2, 16
_NW = _NC * _NS
_BPW = _B // _NW
_SUB = 128
_NSUBT = _BPW // _SUB
_R = 13
_G = 8
_NOUT = _NSUBT // _R

_mesh = plsc.VectorSubcoreMesh(core_axis_name="c", subcore_axis_name="s")


@functools.partial(
    pl.kernel,
    mesh=_mesh,
    compiler_params=pltpu.CompilerParams(use_tc_tiling_on_sc=False),
    out_type=jax.ShapeDtypeStruct((_B, _DIM), jnp.float32),
    scratch_types=[
        pltpu.VMEM((_BPW,), jnp.int32),
        pltpu.VMEM((_R * _SUB, _DIM), jnp.float32),
    ] + [pltpu.SemaphoreType.DMA] * (2 * _R),
)
def _gather_kernel(table_hbm, idx_hbm, out_hbm, idx_v, ring, *sems):
    gsems = sems[:_R]
    ssems = sems[_R:]
    wid = lax.axis_index("s") * _NC + lax.axis_index("c")
    base = wid * _BPW
    pltpu.sync_copy(idx_hbm.at[pl.ds(base, _BPW)], idx_v)

    def slot(s):
        return ring.at[pl.ds(s * _SUB, _SUB)]

    def fire_gather(i, s):
        idx_sl = idx_v.at[pl.ds(i * _SUB, _SUB)]
        pltpu.async_copy(table_hbm.at[idx_sl], slot(s), gsems[s])

    def wait_gather(s):
        pltpu.make_async_copy(
            table_hbm.at[pl.ds(0, _SUB)], slot(s), gsems[s]).wait()

    def fire_store(i, s):
        pltpu.async_copy(slot(s), out_hbm.at[pl.ds(base + i * _SUB, _SUB)],
                         ssems[s])

    def drain_store(s):
        pltpu.make_async_copy(
            slot(s), out_hbm.at[pl.ds(base, _SUB)], ssems[s]).wait()

    for i in range(_G):
        fire_gather(i, i)

    def outer_body(g, carry):
        i0 = g * _R
        for s in range(_R):
            i = i0 + s
            @pl.when(i >= _R - _G)
            def _():
                drain_store((s + _G) % _R)

            @pl.when(i + _G < _NSUBT)
            def _():
                fire_gather(i + _G, (s + _G) % _R)

            wait_gather(s)
            fire_store(i, s)
        return carry

    lax.fori_loop(0, _NOUT, outer_body, 0)

    for j in range(_NSUBT - (_R - _G), _NSUBT):
        drain_store(j % _R)


def kernel(idxs, features):
    flat = idxs.reshape(-1).astype(jnp.int32)
    out = _gather_kernel(features, flat)
    return out.reshape(idxs.shape + (features.shape[1],))
